# Initial kernel scaffold; baseline (speedup 1.0000x reference)
import jax
import jax.numpy as jnp
from jax.experimental import pallas as pl


def kernel(node_features, edge_index, edge_type, store_id, dept_id, cat_id, state_id, item_id, dept_ids, historical_mean, params):
    raise NotImplementedError("write your pallas kernel here")


# speedup vs baseline: 1.0223x; 1.0223x over previous
import numpy as np
import jax
import jax.numpy as jnp
from jax.experimental import pallas as pl

HEADS = 4
HEAD_DIM = 16
T_LEN = 28
C_IN = 8
WINDOWS = (7, 14, 28)
NBLK = 1000


def _build_consts():
    G = np.zeros((T_LEN * C_IN, T_LEN * C_IN), np.float32)
    for i in range(T_LEN * C_IN):
        g0 = (i // C_IN) * C_IN
        G[i, g0:g0 + C_IN] = 1.0 / C_IN
    R8 = np.zeros((8, 64), np.float32)
    T8 = np.zeros((8, 64), np.float32)
    for a in range(8):
        for b in range(8):
            R8[a, a * 8 + b] = 1.0
            T8[b, a * 8 + b] = 1.0
    P1 = np.zeros((64, 128), np.float32)
    P2 = np.zeros((64, 128), np.float32)
    for a in range(8):
        for b in range(8):
            P1[a * 8 + b, a * 16 + b] = 1.0
            P2[a * 8 + b, a * 16 + 8 + b] = 1.0
    return G, R8, T8, P1, P2

_G, _R8, _T8, _P1, _P2 = _build_consts()


def _finite(x):
    return jnp.where(jnp.isfinite(x), x, 0.0)


def _node_kernel(x_ref, idx_ref, item_ref, G_ref, R8_ref, T8_ref, P1_ref,
                 P2_ref, g224_ref, b224_ref, es_ref, ed_ref, ec_ref, est_ref,
                 Wf_ref, bf_ref, gf_ref, bef_ref, out_ref):
    x = x_ref[...]
    G = G_ref[...]
    m = jnp.dot(x, G, preferred_element_type=jnp.float32)
    ex2 = jnp.dot(x * x, G, preferred_element_type=jnp.float32)
    v = ex2 - m * m
    h_in = (x - m) * jax.lax.rsqrt(v + 1e-5) * g224_ref[...] + b224_ref[...]
    h_in = _finite(h_in)

    R8 = R8_ref[...]
    T8 = T8_ref[...]
    P1 = P1_ref[...]
    P2 = P2_ref[...]

    def xt(u):
        return h_in[:, u * 8:(u + 1) * 8]

    B = x.shape[0]
    P = jnp.zeros((B, 64), jnp.float32)
    Q = jnp.zeros((B, 64), jnp.float32)
    caps = {}
    for u in range(T_LEN - 2, -1, -1):
        d_u = xt(u + 1) - xt(u)
        duT = jnp.dot(d_u, T8, preferred_element_type=jnp.float32)
        P = P + jnp.dot(xt(u), R8, preferred_element_type=jnp.float32) * duT
        Q = Q + jnp.dot(d_u, R8, preferred_element_type=jnp.float32) * duT
        if (T_LEN - u) in [w for w in WINDOWS]:
            caps[T_LEN - u] = (P, Q)

    feats = []
    xlast = xt(T_LEN - 1)
    for w in WINDOWS:
        Pw, Qw = caps[w]
        x0 = xt(T_LEN - w)
        S = xlast - x0
        A = Pw - jnp.dot(x0, R8, preferred_element_type=jnp.float32) * \
            jnp.dot(S, T8, preferred_element_type=jnp.float32)
        M1 = A + 0.5 * Qw
        M2 = A + Qw
        top = jnp.dot(M1, P1, preferred_element_type=jnp.float32) + \
            jnp.dot(M2, P2, preferred_element_type=jnp.float32)
        bot = jnp.dot(A, P1, preferred_element_type=jnp.float32) + \
            jnp.dot(M1, P2, preferred_element_type=jnp.float32)
        feats.append(S)
        feats.append(S)
        feats.append(top)
        feats.append(bot)
    sig = _finite(jnp.concatenate(feats, axis=1))

    idx = idx_ref[...]
    embs = []
    for j, (vocab, tbl) in enumerate(((10, es_ref), (7, ed_ref),
                                      (3, ec_ref), (3, est_ref))):
        ii = jnp.clip(idx[:, j:j + 1], 0, vocab - 1)
        oh = (ii == jax.lax.broadcasted_iota(jnp.int32, (1, tbl.shape[0]), 1)
              ).astype(jnp.float32)
        embs.append(jnp.dot(oh, tbl[...], preferred_element_type=jnp.float32))
    embs.append(item_ref[...])
    h = jnp.concatenate([sig] + embs, axis=1)

    f = jnp.dot(h, Wf_ref[...], preferred_element_type=jnp.float32) + bf_ref[...]
    f = jax.nn.gelu(f)
    mu = jnp.mean(f, axis=1, keepdims=True)
    va = jnp.mean(f * f, axis=1, keepdims=True) - mu * mu
    f = (f - mu) * jax.lax.rsqrt(va + 1e-5) * gf_ref[...] + bef_ref[...]
    out_ref[...] = _finite(f)


def _head_kernel(h_ref, did_ref, hist_ref, W1_ref, b1_ref, gp_ref, bep_ref,
                 W2_ref, b2_ref, scale_ref, gs_ref, out_ref):
    h = h_ref[...]
    z = jnp.dot(h, W1_ref[...], preferred_element_type=jnp.float32) + b1_ref[...]
    z = jax.nn.gelu(z)
    mu = jnp.mean(z, axis=1, keepdims=True)
    va = jnp.mean(z * z, axis=1, keepdims=True) - mu * mu
    z = (z - mu) * jax.lax.rsqrt(va + 1e-5) * gp_ref[...] + bep_ref[...]
    out = jnp.dot(z, W2_ref[...], preferred_element_type=jnp.float32) + b2_ref[...]
    preds = _finite(out * scale_ref[...])
    did = did_ref[...]
    oh = (did == jax.lax.broadcasted_iota(jnp.int32, (1, 8), 1)
          ).astype(jnp.float32)
    gsd = jnp.dot(oh, gs_ref[...], preferred_element_type=jnp.float32)
    preds = preds * gsd
    preds = jnp.minimum(preds, hist_ref[...] * 20.0)
    out_ref[...] = jnp.clip(preds, 0.0, 1000.0)


def _rep(shape):
    return pl.BlockSpec(shape, lambda i: tuple(0 for _ in shape))


def _gat_layer(h, src, dst, etype, W, a_src, a_dst, tb):
    n = h.shape[0]
    hp = (h @ W).reshape(n, HEADS, HEAD_DIM)
    als = (hp * a_src[None]).sum(-1)
    ald = (hp * a_dst[None]).sum(-1)
    e = jax.nn.leaky_relu(als[src] + ald[dst] + tb[etype], 0.2)
    m = jax.ops.segment_max(e, dst, num_segments=n)
    m = jnp.where(jnp.isfinite(m), m, 0.0)
    ex = jnp.exp(e - m[dst])
    den = jax.ops.segment_sum(ex, dst, num_segments=n)
    attn = ex / (den[dst] + 1e-9)
    out = jax.ops.segment_sum(hp[src] * attn[..., None], dst,
                              num_segments=n).reshape(n, HEADS * HEAD_DIM)
    return jax.nn.elu(out) + h


def kernel(node_features, edge_index, edge_type, store_id, dept_id, cat_id,
           state_id, item_id, dept_ids, historical_mean, params):
    p = params
    n = node_features.shape[0]
    x224 = node_features.reshape(n, T_LEN * C_IN)
    idx4 = jnp.stack([store_id, dept_id, cat_id, state_id], axis=1
                     ).astype(jnp.int32)
    item_e = jnp.take(p['emb_item'],
                      jnp.clip(item_id, 0, p['emb_item'].shape[0] - 1), axis=0)
    g224 = jnp.tile(p['ln_g'], T_LEN)[None, :]
    b224 = jnp.tile(p['ln_b'], T_LEN)[None, :]

    grid = n // NBLK
    h0 = pl.pallas_call(
        _node_kernel,
        grid=(grid,),
        in_specs=[
            pl.BlockSpec((NBLK, T_LEN * C_IN), lambda i: (i, 0)),
            pl.BlockSpec((NBLK, 4), lambda i: (i, 0)),
            pl.BlockSpec((NBLK, 16), lambda i: (i, 0)),
            _rep(_G.shape), _rep(_R8.shape), _rep(_T8.shape),
            _rep(_P1.shape), _rep(_P2.shape),
            _rep((1, T_LEN * C_IN)), _rep((1, T_LEN * C_IN)),
            _rep(p['emb_store'].shape), _rep(p['emb_dept'].shape),
            _rep(p['emb_cat'].shape), _rep(p['emb_state'].shape),
            _rep(p['W_fuse'].shape), _rep((1, 64)), _rep((1, 64)),
            _rep((1, 64)),
        ],
        out_specs=pl.BlockSpec((NBLK, 64), lambda i: (i, 0)),
        out_shape=jax.ShapeDtypeStruct((n, 64), jnp.float32),
    )(x224, idx4, item_e, jnp.asarray(_G), jnp.asarray(_R8), jnp.asarray(_T8),
      jnp.asarray(_P1), jnp.asarray(_P2), g224, b224,
      p['emb_store'], p['emb_dept'], p['emb_cat'], p['emb_state'],
      p['W_fuse'], p['b_fuse'][None, :], p['g_fuse'][None, :],
      p['be_fuse'][None, :])

    src, dst = edge_index[0], edge_index[1]
    h = _gat_layer(h0, src, dst, edge_type, p['gat_W0'], p['gat_as0'],
                   p['gat_ad0'], p['gat_tb0'])
    h = _gat_layer(h, src, dst, edge_type, p['gat_W1'], p['gat_as1'],
                   p['gat_ad1'], p['gat_tb1'])
    h = _finite(h)

    scale = jnp.clip(jax.nn.softplus(p['hscale']), 0.1, 5.0)[None, :]
    gs = jnp.clip(jax.nn.softplus(p['group_scale']), 1.0 / 20.0, 20.0)
    gs8 = jnp.pad(gs, (0, 1))[:, None]
    did = dept_ids.astype(jnp.int32)[:, None]

    preds = pl.pallas_call(
        _head_kernel,
        grid=(grid,),
        in_specs=[
            pl.BlockSpec((NBLK, 64), lambda i: (i, 0)),
            pl.BlockSpec((NBLK, 1), lambda i: (i, 0)),
            pl.BlockSpec((NBLK, 1), lambda i: (i, 0)),
            _rep(p['Wp1'].shape), _rep((1, 128)), _rep((1, 128)),
            _rep((1, 128)), _rep(p['Wp2'].shape), _rep((1, 28)),
            _rep((1, 28)), _rep((8, 1)),
        ],
        out_specs=pl.BlockSpec((NBLK, 28), lambda i: (i, 0)),
        out_shape=jax.ShapeDtypeStruct((n, 28), jnp.float32),
    )(h, did, historical_mean, p['Wp1'], p['bp1'][None, :], p['gp'][None, :],
      p['bep'][None, :], p['Wp2'], p['bp2'][None, :], scale, gs8)
    return preds
